# baseline (speedup 1.0000x reference)
import jax
import jax.numpy as jnp
from jax.experimental import pallas as pl
from jax.experimental.pallas import tpu as pltpu

_S_BLK = 1024


def _bind_kernel(x_ref, p_ref, o_ref):
    o_ref[...] = x_ref[...] * p_ref[...]


def kernel(input_tensor, position_vectors):
    bsz, seq_len, d = input_tensor.shape
    pos = position_vectors[:seq_len, :d]
    grid = (seq_len // _S_BLK, bsz)
    return pl.pallas_call(
        _bind_kernel,
        grid=grid,
        in_specs=[
            pl.BlockSpec((1, _S_BLK, d), lambda s, b: (b, s, 0)),
            pl.BlockSpec((_S_BLK, d), lambda s, b: (s, 0)),
        ],
        out_specs=pl.BlockSpec((1, _S_BLK, d), lambda s, b: (b, s, 0)),
        out_shape=jax.ShapeDtypeStruct((bsz, seq_len, d), input_tensor.dtype),
        compiler_params=pltpu.CompilerParams(
            dimension_semantics=("parallel", "parallel")),
    )(input_tensor, pos)


# speedup vs baseline: 1.0027x; 1.0027x over previous
import jax
import jax.numpy as jnp
from jax.experimental import pallas as pl

_S_BLK = 1024


def _bind_kernel(x_ref, p_ref, o_ref):
    o_ref[...] = x_ref[...] * p_ref[...]


def kernel(input_tensor, position_vectors):
    bsz, seq_len, d = input_tensor.shape
    pos = position_vectors[:seq_len, :d]
    grid = (seq_len // _S_BLK, bsz)
    return pl.pallas_call(
        _bind_kernel,
        grid=grid,
        in_specs=[
            pl.BlockSpec((1, _S_BLK, d), lambda s, b: (b, s, 0)),
            pl.BlockSpec((_S_BLK, d), lambda s, b: (s, 0)),
        ],
        out_specs=pl.BlockSpec((1, _S_BLK, d), lambda s, b: (b, s, 0)),
        out_shape=jax.ShapeDtypeStruct((bsz, seq_len, d), input_tensor.dtype),
    )(input_tensor, pos)
